# baseline (speedup 1.0000x reference)
import math

import jax
import jax.numpy as jnp
import numpy as np
from jax import lax
from jax.experimental import pallas as pl
from jax.experimental.pallas import tpu as pltpu
from jax.experimental.pallas import tpu_sc as plsc

D_MODEL = 128
MAX_LEN = 1500
VOCAB = 28
BATCH = 256
SEQ = 1024

NC, NS, LANES = 2, 16, 16
NW = NC * NS
TOK_PER_W = BATCH * SEQ // NW
QUARTERS = 4
Q = SEQ // QUARTERS
QD = Q * D_MODEL
SEQ_PER_W = TOK_PER_W // SEQ


def _make_pe_np(max_len, d_model):
    position = np.arange(0, max_len, dtype=np.float32)[:, None]
    div_term = np.exp(
        np.arange(0, d_model, 2).astype(np.float32) * -(math.log(10000.0) / d_model)
    )
    pe = np.zeros((max_len, d_model), dtype=np.float32)
    pe[:, 0::2] = np.sin(position * div_term)
    pe[:, 1::2] = np.cos(position * div_term)
    return pe


_PE_NP = _make_pe_np(MAX_LEN, D_MODEL)[:SEQ].reshape(-1)


def _sc_embed(tokens_flat, table_flat, pe_flat):
    mesh = plsc.VectorSubcoreMesh(
        core_axis_name="c", subcore_axis_name="s", num_cores=NC, num_subcores=NS
    )

    def body(tok_hbm, table_hbm, pe_hbm, out_hbm,
             table_v, pe_v, idx_v, rows0, rows1, sem0, sem1):
        wid = lax.axis_index("s") * NC + lax.axis_index("c")
        base = wid * TOK_PER_W
        pltpu.sync_copy(table_hbm, table_v)
        pltpu.sync_copy(tok_hbm.at[pl.ds(base, TOK_PER_W)], idx_v)
        lanes128 = lax.broadcasted_iota(jnp.int32, (LANES,), 0) * D_MODEL
        rows = (rows0, rows1)
        sems = (sem0, sem1)

        def compute_chunk(loc, rows_b):
            @plsc.parallel_loop(0, Q // LANES, unroll=2)
            def _rb_body(rb):
                tokv = idx_v[pl.ds(loc + rb * LANES, LANES)] * D_MODEL
                gbase = rb * (LANES * D_MODEL)
                for lane in range(LANES):
                    tbase = tokv[lane]
                    rbase = gbase + lane * D_MODEL
                    for j in range(D_MODEL // LANES):
                        tv = table_v[pl.ds(tbase + j * LANES, LANES)]
                        pv = pe_v[pl.ds(rbase + j * LANES, LANES)]
                        rows_b[pl.ds(rbase + j * LANES, LANES)] = tv + pv

        def q_body(q, _):
            pltpu.sync_copy(pe_hbm.at[pl.ds(q * QD, QD)], pe_v)

            def s2_body(s2, _):
                for b in range(2):
                    s = s2 * 2 + b
                    g = base + s * SEQ + q * Q

                    @pl.when(jnp.logical_or(q > 0, s2 > 0))
                    def _wait(b=b):
                        pltpu.make_async_copy(
                            rows[b], out_hbm.at[pl.ds(0, QD)], sems[b]
                        ).wait()

                    compute_chunk(s * SEQ + q * Q, rows[b])
                    pltpu.async_copy(
                        rows[b], out_hbm.at[pl.ds(g * D_MODEL, QD)], sems[b]
                    )
                return 0

            lax.fori_loop(0, SEQ_PER_W // 2, s2_body, 0)
            return 0

        lax.fori_loop(0, QUARTERS, q_body, 0)
        for b in range(2):
            pltpu.make_async_copy(
                rows[b], out_hbm.at[pl.ds(0, QD)], sems[b]
            ).wait()

    run = pl.kernel(
        body,
        out_type=jax.ShapeDtypeStruct((BATCH * SEQ * D_MODEL,), jnp.float32),
        mesh=mesh,
        compiler_params=pltpu.CompilerParams(needs_layout_passes=False),
        scratch_types=[
            pltpu.VMEM((VOCAB * D_MODEL,), jnp.float32),
            pltpu.VMEM((QD,), jnp.float32),
            pltpu.VMEM((TOK_PER_W,), jnp.int32),
            pltpu.VMEM((QD,), jnp.float32),
            pltpu.VMEM((QD,), jnp.float32),
            pltpu.SemaphoreType.DMA,
            pltpu.SemaphoreType.DMA,
        ],
    )
    return run(tokens_flat, table_flat, pe_flat)


def kernel(tokens, table):
    tokens_flat = tokens.reshape(-1).astype(jnp.int32)
    out = _sc_embed(tokens_flat, table.reshape(-1), jnp.asarray(_PE_NP))
    return out.reshape(BATCH, SEQ, D_MODEL)


# speedup vs baseline: 1.2222x; 1.2222x over previous
import math

import jax
import jax.numpy as jnp
import numpy as np
from jax import lax
from jax.experimental import pallas as pl
from jax.experimental.pallas import tpu as pltpu
from jax.experimental.pallas import tpu_sc as plsc

D_MODEL = 128
MAX_LEN = 1500
VOCAB = 28
BATCH = 256
SEQ = 1024

NC, NS, LANES = 2, 16, 16
NW = NC * NS
POS_PER_W = SEQ // NW
SUB = 16
NSUB = POS_PER_W // SUB
SG = 8
NCHUNK = BATCH // SG
ROW_W = SUB * D_MODEL


def _make_pe_np(max_len, d_model):
    position = np.arange(0, max_len, dtype=np.float32)[:, None]
    div_term = np.exp(
        np.arange(0, d_model, 2).astype(np.float32) * -(math.log(10000.0) / d_model)
    )
    pe = np.zeros((max_len, d_model), dtype=np.float32)
    pe[:, 0::2] = np.sin(position * div_term)
    pe[:, 1::2] = np.cos(position * div_term)
    return pe


_PE_NP = _make_pe_np(MAX_LEN, D_MODEL)[:SEQ].reshape(-1)


def _sc_embed(tokens, table_flat, pe_flat):
    mesh = plsc.VectorSubcoreMesh(
        core_axis_name="c", subcore_axis_name="s", num_cores=NC, num_subcores=NS
    )

    def body(tok_hbm, table_hbm, pe_hbm, out_hbm,
             table_v, pe_v, tok_v, comb_v, rows0, rows1, sem0, sem1):
        wid = lax.axis_index("s") * NC + lax.axis_index("c")
        lw = wid * POS_PER_W
        pltpu.sync_copy(table_hbm, table_v)
        iota = lax.broadcasted_iota(jnp.int32, (LANES,), 0)
        rows = (rows0, rows1)
        sems = (sem0, sem1)

        for sb in range(NSUB):
            l0 = lw + sb * SUB
            pltpu.sync_copy(pe_hbm.at[pl.ds(l0 * D_MODEL, SUB * D_MODEL)], pe_v)
            g = wid * NSUB + sb
            pltpu.sync_copy(tok_hbm.at[pl.ds(g * (BATCH * SUB), BATCH * SUB)], tok_v)

            @plsc.parallel_loop(0, VOCAB * SUB, unroll=1)
            def _build(i):
                ta = (i >> 4) * D_MODEL
                pa = (i & (SUB - 1)) * D_MODEL
                ca = i * D_MODEL
                for j in range(D_MODEL // LANES):
                    tv = table_v[pl.ds(ta + j * LANES, LANES)]
                    pv = pe_v[pl.ds(pa + j * LANES, LANES)]
                    comb_v[pl.ds(ca + j * LANES, LANES)] = tv + pv

            def pair_body(pair, _, sb=sb, l0=l0):
                for b in range(2):
                    chunk = pair * 2 + b
                    s0 = chunk * SG

                    if sb == 0:
                        @pl.when(pair > 0)
                        def _wait(b=b):
                            pltpu.make_async_copy(
                                rows[b],
                                out_hbm.at[pl.ds(0, SG), pl.ds(0, ROW_W)],
                                sems[b],
                            ).wait()
                    else:
                        pltpu.make_async_copy(
                            rows[b],
                            out_hbm.at[pl.ds(0, SG), pl.ds(0, ROW_W)],
                            sems[b],
                        ).wait()

                    rows_b = rows[b]

                    @plsc.parallel_loop(0, SG, unroll=1)
                    def _seq(s):
                        tokv = tok_v[pl.ds((s0 + s) * SUB, SUB)]
                        cvec = (tokv << 11) + (iota << 7)
                        for p in range(SUB):
                            cb = cvec[p]
                            for j in range(D_MODEL // LANES):
                                rows_b[s, pl.ds(p * D_MODEL + j * LANES, LANES)] = (
                                    comb_v[pl.ds(cb + j * LANES, LANES)]
                                )

                    pltpu.async_copy(
                        rows_b,
                        out_hbm.at[pl.ds(s0, SG), pl.ds(l0 * D_MODEL, ROW_W)],
                        sems[b],
                    )
                return 0

            lax.fori_loop(0, NCHUNK // 2, pair_body, 0)

        for b in range(2):
            pltpu.make_async_copy(
                rows[b], out_hbm.at[pl.ds(0, SG), pl.ds(0, ROW_W)], sems[b]
            ).wait()

    run = pl.kernel(
        body,
        out_type=jax.ShapeDtypeStruct((BATCH, SEQ * D_MODEL), jnp.float32),
        mesh=mesh,
        compiler_params=pltpu.CompilerParams(needs_layout_passes=False),
        scratch_types=[
            pltpu.VMEM((VOCAB * D_MODEL,), jnp.float32),
            pltpu.VMEM((SUB * D_MODEL,), jnp.float32),
            pltpu.VMEM((BATCH * SUB,), jnp.int32),
            pltpu.VMEM((VOCAB * SUB * D_MODEL,), jnp.float32),
            pltpu.VMEM((SG, ROW_W), jnp.float32),
            pltpu.VMEM((SG, ROW_W), jnp.float32),
            pltpu.SemaphoreType.DMA,
            pltpu.SemaphoreType.DMA,
        ],
    )
    return run(tokens, table_flat, pe_flat)


def kernel(tokens, table):
    tok_re = (
        tokens.astype(jnp.int32)
        .reshape(BATCH, SEQ // SUB, SUB)
        .transpose(1, 0, 2)
        .reshape(-1)
    )
    out = _sc_embed(tok_re, table.reshape(-1), jnp.asarray(_PE_NP))
    return out.reshape(BATCH, SEQ, D_MODEL)


# speedup vs baseline: 2.4392x; 1.9958x over previous
import math

import jax
import jax.numpy as jnp
import numpy as np
from jax import lax
from jax.experimental import pallas as pl
from jax.experimental.pallas import tpu as pltpu
from jax.experimental.pallas import tpu_sc as plsc

D_MODEL = 128
MAX_LEN = 1500
VOCAB = 28
BATCH = 256
SEQ = 1024

NC, NS, LANES = 2, 16, 16
NW = NC * NS
TOK_PER_W = BATCH * SEQ // NW
QUARTERS = 4
Q = SEQ // QUARTERS
QD = Q * D_MODEL
SEQ_PER_W = TOK_PER_W // SEQ


def _make_pe_np(max_len, d_model):
    position = np.arange(0, max_len, dtype=np.float32)[:, None]
    div_term = np.exp(
        np.arange(0, d_model, 2).astype(np.float32) * -(math.log(10000.0) / d_model)
    )
    pe = np.zeros((max_len, d_model), dtype=np.float32)
    pe[:, 0::2] = np.sin(position * div_term)
    pe[:, 1::2] = np.cos(position * div_term)
    return pe


def _interleave32_np(x2d):
    n = x2d.shape[0]
    g = x2d.reshape(n, D_MODEL // 32, 2, 16)
    m = np.empty((n, D_MODEL // 32, 16, 2), dtype=x2d.dtype)
    m[..., 0] = g[:, :, 0, :]
    m[..., 1] = g[:, :, 1, :]
    return m.reshape(n * D_MODEL)


_PE_NP = _interleave32_np(_make_pe_np(MAX_LEN, D_MODEL)[:SEQ])


def _sc_embed(tokens_flat, table_bf, pe_bf):
    mesh = plsc.VectorSubcoreMesh(
        core_axis_name="c", subcore_axis_name="s", num_cores=NC, num_subcores=NS
    )

    def body(tok_hbm, table_hbm, pe_hbm, out_hbm,
             table_v, pe_v, idx_v, rows0, rows1, sem0, sem1):
        wid = lax.axis_index("s") * NC + lax.axis_index("c")
        base = wid * TOK_PER_W
        pltpu.sync_copy(table_hbm, table_v)
        pltpu.sync_copy(tok_hbm.at[pl.ds(base, TOK_PER_W)], idx_v)
        rows = (rows0, rows1)
        sems = (sem0, sem1)

        def compute_chunk(loc, rows_b):
            @plsc.parallel_loop(0, Q // LANES, unroll=1)
            def _rb_body(rb):
                tokv = idx_v[pl.ds(loc + rb * LANES, LANES)] * (D_MODEL // 2)
                gbase = rb * (LANES * D_MODEL)
                for lane in range(LANES):
                    tbase = tokv[lane]
                    rbase = gbase + lane * D_MODEL
                    pbase = (gbase // 2) + lane * (D_MODEL // 2)
                    for j in range(D_MODEL // 32):
                        tvi = table_v[pl.ds(tbase + j * LANES, LANES)]
                        pvi = pe_v[pl.ds(pbase + j * LANES, LANES)]
                        t0, t1 = plsc.unpack(
                            plsc.bitcast(tvi, jnp.bfloat16),
                            format=plsc.PackFormat.INTERLEAVED,
                            preferred_element_type=jnp.float32,
                        )
                        p0, p1 = plsc.unpack(
                            plsc.bitcast(pvi, jnp.bfloat16),
                            format=plsc.PackFormat.INTERLEAVED,
                            preferred_element_type=jnp.float32,
                        )
                        off = rbase + j * 32
                        rows_b[pl.ds(off, LANES)] = t0 + p0
                        rows_b[pl.ds(off + LANES, LANES)] = t1 + p1

        def q_body(q, _):
            pltpu.sync_copy(pe_hbm.at[pl.ds(q * (QD // 2), QD // 2)], pe_v)

            def s2_body(s2, _):
                for b in range(2):
                    s = s2 * 2 + b
                    g = base + s * SEQ + q * Q

                    @pl.when(jnp.logical_or(q > 0, s2 > 0))
                    def _wait(b=b):
                        pltpu.make_async_copy(
                            rows[b], out_hbm.at[pl.ds(0, QD)], sems[b]
                        ).wait()

                    compute_chunk(s * SEQ + q * Q, rows[b])
                    pltpu.async_copy(
                        rows[b], out_hbm.at[pl.ds(g * D_MODEL, QD)], sems[b]
                    )
                return 0

            lax.fori_loop(0, SEQ_PER_W // 2, s2_body, 0)
            return 0

        lax.fori_loop(0, QUARTERS, q_body, 0)
        for b in range(2):
            pltpu.make_async_copy(
                rows[b], out_hbm.at[pl.ds(0, QD)], sems[b]
            ).wait()

    run = pl.kernel(
        body,
        out_type=jax.ShapeDtypeStruct((BATCH * SEQ * D_MODEL,), jnp.float32),
        mesh=mesh,
        compiler_params=pltpu.CompilerParams(needs_layout_passes=False),
        scratch_types=[
            pltpu.VMEM((VOCAB * D_MODEL // 2,), jnp.int32),
            pltpu.VMEM((QD // 2,), jnp.int32),
            pltpu.VMEM((TOK_PER_W,), jnp.int32),
            pltpu.VMEM((QD,), jnp.float32),
            pltpu.VMEM((QD,), jnp.float32),
            pltpu.SemaphoreType.DMA,
            pltpu.SemaphoreType.DMA,
        ],
    )
    return run(tokens_flat, table_bf, pe_bf)


def kernel(tokens, table):
    tokens_flat = tokens.reshape(-1).astype(jnp.int32)
    table_il = (
        table.reshape(VOCAB, D_MODEL // 32, 2, 16)
        .transpose(0, 1, 3, 2)
        .reshape(-1)
    )
    table_bf = table_il.astype(jnp.bfloat16)
    pe_bf = jnp.asarray(_PE_NP).astype(jnp.bfloat16)
    table_i = lax.bitcast_convert_type(table_bf.reshape(-1, 2), jnp.int32)
    pe_i = lax.bitcast_convert_type(pe_bf.reshape(-1, 2), jnp.int32)
    out = _sc_embed(tokens_flat, table_i, pe_i)
    return out.reshape(BATCH, SEQ, D_MODEL)


# speedup vs baseline: 2.5310x; 1.0376x over previous
import math

import jax
import jax.numpy as jnp
import numpy as np
from jax import lax
from jax.experimental import pallas as pl
from jax.experimental.pallas import tpu as pltpu
from jax.experimental.pallas import tpu_sc as plsc

D_MODEL = 128
MAX_LEN = 1500
VOCAB = 28
BATCH = 256
SEQ = 1024

NC, NS, LANES = 2, 16, 16
NW = NC * NS
TOK_PER_W = BATCH * SEQ // NW
QUARTERS = 4
Q = SEQ // QUARTERS
QD = Q * D_MODEL
SEQ_PER_W = TOK_PER_W // SEQ


def _make_pe_np(max_len, d_model):
    position = np.arange(0, max_len, dtype=np.float32)[:, None]
    div_term = np.exp(
        np.arange(0, d_model, 2).astype(np.float32) * -(math.log(10000.0) / d_model)
    )
    pe = np.zeros((max_len, d_model), dtype=np.float32)
    pe[:, 0::2] = np.sin(position * div_term)
    pe[:, 1::2] = np.cos(position * div_term)
    return pe


def _interleave32_np(x2d):
    n = x2d.shape[0]
    g = x2d.reshape(n, D_MODEL // 32, 2, 16)
    m = np.empty((n, D_MODEL // 32, 16, 2), dtype=x2d.dtype)
    m[..., 0] = g[:, :, 0, :]
    m[..., 1] = g[:, :, 1, :]
    return m.reshape(n * D_MODEL)


_PE_NP = _interleave32_np(_make_pe_np(MAX_LEN, D_MODEL)[:SEQ])


def _sc_embed(tokens_flat, table_bf, pe_bf):
    mesh = plsc.VectorSubcoreMesh(
        core_axis_name="c", subcore_axis_name="s", num_cores=NC, num_subcores=NS
    )

    def body(tok_hbm, table_hbm, pe_hbm, out_hbm,
             table_v, pe_v, idx_v, rows0, rows1, sem0, sem1):
        wid = lax.axis_index("s") * NC + lax.axis_index("c")
        base = wid * TOK_PER_W
        pltpu.sync_copy(table_hbm, table_v)
        pltpu.sync_copy(tok_hbm.at[pl.ds(base, TOK_PER_W)], idx_v)
        rows = (rows0, rows1)
        sems = (sem0, sem1)

        def compute_chunk(loc, rows_b):
            @plsc.parallel_loop(0, Q // LANES, unroll=1)
            def _rb_body(rb):
                tokv = idx_v[pl.ds(loc + rb * LANES, LANES)] * (D_MODEL // 2)
                gbase = rb * (LANES * D_MODEL)
                for lane in range(LANES):
                    tbase = tokv[lane]
                    rbase = gbase + lane * D_MODEL
                    pbase = (gbase // 2) + lane * (D_MODEL // 2)
                    for j in range(D_MODEL // 32):
                        tvi = table_v[pl.ds(tbase + j * LANES, LANES)]
                        pvi = pe_v[pl.ds(pbase + j * LANES, LANES)]
                        sb = plsc.bitcast(tvi, jnp.bfloat16) + plsc.bitcast(
                            pvi, jnp.bfloat16
                        )
                        s0, s1 = plsc.unpack(
                            sb,
                            format=plsc.PackFormat.INTERLEAVED,
                            preferred_element_type=jnp.float32,
                        )
                        off = rbase + j * 32
                        rows_b[pl.ds(off, LANES)] = s0
                        rows_b[pl.ds(off + LANES, LANES)] = s1

        def q_body(q, _):
            pltpu.sync_copy(pe_hbm.at[pl.ds(q * (QD // 2), QD // 2)], pe_v)

            def s2_body(s2, _):
                for b in range(2):
                    s = s2 * 2 + b
                    g = base + s * SEQ + q * Q

                    @pl.when(jnp.logical_or(q > 0, s2 > 0))
                    def _wait(b=b):
                        pltpu.make_async_copy(
                            rows[b], out_hbm.at[pl.ds(0, QD)], sems[b]
                        ).wait()

                    compute_chunk(s * SEQ + q * Q, rows[b])
                    pltpu.async_copy(
                        rows[b], out_hbm.at[pl.ds(g * D_MODEL, QD)], sems[b]
                    )
                return 0

            lax.fori_loop(0, SEQ_PER_W // 2, s2_body, 0)
            return 0

        lax.fori_loop(0, QUARTERS, q_body, 0)
        for b in range(2):
            pltpu.make_async_copy(
                rows[b], out_hbm.at[pl.ds(0, QD)], sems[b]
            ).wait()

    run = pl.kernel(
        body,
        out_type=jax.ShapeDtypeStruct((BATCH * SEQ * D_MODEL,), jnp.float32),
        mesh=mesh,
        compiler_params=pltpu.CompilerParams(needs_layout_passes=False),
        scratch_types=[
            pltpu.VMEM((VOCAB * D_MODEL // 2,), jnp.int32),
            pltpu.VMEM((QD // 2,), jnp.int32),
            pltpu.VMEM((TOK_PER_W,), jnp.int32),
            pltpu.VMEM((QD,), jnp.float32),
            pltpu.VMEM((QD,), jnp.float32),
            pltpu.SemaphoreType.DMA,
            pltpu.SemaphoreType.DMA,
        ],
    )
    return run(tokens_flat, table_bf, pe_bf)


def kernel(tokens, table):
    tokens_flat = tokens.reshape(-1).astype(jnp.int32)
    table_il = (
        table.reshape(VOCAB, D_MODEL // 32, 2, 16)
        .transpose(0, 1, 3, 2)
        .reshape(-1)
    )
    table_bf = table_il.astype(jnp.bfloat16)
    pe_bf = jnp.asarray(_PE_NP).astype(jnp.bfloat16)
    table_i = lax.bitcast_convert_type(table_bf.reshape(-1, 2), jnp.int32)
    pe_i = lax.bitcast_convert_type(pe_bf.reshape(-1, 2), jnp.int32)
    out = _sc_embed(tokens_flat, table_i, pe_i)
    return out.reshape(BATCH, SEQ, D_MODEL)


# speedup vs baseline: 3.2920x; 1.3007x over previous
import math

import jax
import jax.numpy as jnp
import numpy as np
from jax import lax
from jax.experimental import pallas as pl
from jax.experimental.pallas import tpu as pltpu
from jax.experimental.pallas import tpu_sc as plsc

D_MODEL = 128
MAX_LEN = 1500
VOCAB = 28
BATCH = 256
SEQ = 1024

NC, NS, LANES = 2, 16, 16
NW = NC * NS
TOK_PER_W = BATCH * SEQ // NW
QUARTERS = 4
Q = SEQ // QUARTERS
QD = Q * D_MODEL
SEQ_PER_W = TOK_PER_W // SEQ


def _make_pe_np(max_len, d_model):
    position = np.arange(0, max_len, dtype=np.float32)[:, None]
    div_term = np.exp(
        np.arange(0, d_model, 2).astype(np.float32) * -(math.log(10000.0) / d_model)
    )
    pe = np.zeros((max_len, d_model), dtype=np.float32)
    pe[:, 0::2] = np.sin(position * div_term)
    pe[:, 1::2] = np.cos(position * div_term)
    return pe


def _interleave32_np(x2d):
    n = x2d.shape[0]
    g = x2d.reshape(n, D_MODEL // 32, 2, 16)
    m = np.empty((n, D_MODEL // 32, 16, 2), dtype=x2d.dtype)
    m[..., 0] = g[:, :, 0, :]
    m[..., 1] = g[:, :, 1, :]
    return m.reshape(n * D_MODEL)


_PE_NP = _interleave32_np(_make_pe_np(MAX_LEN, D_MODEL)[:SEQ])


def _sc_embed(tokens_flat, table_bf, pe_bf):
    mesh = plsc.VectorSubcoreMesh(
        core_axis_name="c", subcore_axis_name="s", num_cores=NC, num_subcores=NS
    )

    def body(tok_hbm, table_hbm, pe_hbm, out_hbm,
             table_v, pe_v, idx_v, rows0, rows1, sem0, sem1):
        wid = lax.axis_index("s") * NC + lax.axis_index("c")
        base = wid * TOK_PER_W
        pltpu.sync_copy(table_hbm, table_v)
        pltpu.sync_copy(tok_hbm.at[pl.ds(base, TOK_PER_W)], idx_v)
        rows = (rows0, rows1)
        sems = (sem0, sem1)

        def compute_chunk(loc, rows_b):
            @plsc.parallel_loop(0, Q // LANES, unroll=1)
            def _rb_body(rb):
                tokv = idx_v[pl.ds(loc + rb * LANES, LANES)] * (D_MODEL // 2)
                gbase = rb * (LANES * D_MODEL)
                nj = D_MODEL // 32
                for lane in range(LANES):
                    tbase = tokv[lane]
                    rbase = gbase + lane * D_MODEL
                    pbase = (gbase // 2) + lane * (D_MODEL // 2)
                    tvi = [table_v[pl.ds(tbase + j * LANES, LANES)] for j in range(nj)]
                    pvi = [pe_v[pl.ds(pbase + j * LANES, LANES)] for j in range(nj)]
                    sums = [
                        plsc.bitcast(tvi[j], jnp.bfloat16)
                        + plsc.bitcast(pvi[j], jnp.bfloat16)
                        for j in range(nj)
                    ]
                    for j in range(nj):
                        s0, s1 = plsc.unpack(
                            sums[j],
                            format=plsc.PackFormat.INTERLEAVED,
                            preferred_element_type=jnp.float32,
                        )
                        off = rbase + j * 32
                        rows_b[pl.ds(off, LANES)] = s0
                        rows_b[pl.ds(off + LANES, LANES)] = s1

        def q_body(q, _):
            pltpu.sync_copy(pe_hbm.at[pl.ds(q * (QD // 2), QD // 2)], pe_v)

            def s2_body(s2, _):
                for b in range(2):
                    s = s2 * 2 + b
                    g = base + s * SEQ + q * Q

                    @pl.when(jnp.logical_or(q > 0, s2 > 0))
                    def _wait(b=b):
                        pltpu.make_async_copy(
                            rows[b], out_hbm.at[pl.ds(0, QD)], sems[b]
                        ).wait()

                    compute_chunk(s * SEQ + q * Q, rows[b])
                    pltpu.async_copy(
                        rows[b], out_hbm.at[pl.ds(g * D_MODEL, QD)], sems[b]
                    )
                return 0

            lax.fori_loop(0, SEQ_PER_W // 2, s2_body, 0)
            return 0

        lax.fori_loop(0, QUARTERS, q_body, 0)
        for b in range(2):
            pltpu.make_async_copy(
                rows[b], out_hbm.at[pl.ds(0, QD)], sems[b]
            ).wait()

    run = pl.kernel(
        body,
        out_type=jax.ShapeDtypeStruct((BATCH * SEQ * D_MODEL,), jnp.float32),
        mesh=mesh,
        compiler_params=pltpu.CompilerParams(needs_layout_passes=False),
        scratch_types=[
            pltpu.VMEM((VOCAB * D_MODEL // 2,), jnp.int32),
            pltpu.VMEM((QD // 2,), jnp.int32),
            pltpu.VMEM((TOK_PER_W,), jnp.int32),
            pltpu.VMEM((QD,), jnp.float32),
            pltpu.VMEM((QD,), jnp.float32),
            pltpu.SemaphoreType.DMA,
            pltpu.SemaphoreType.DMA,
        ],
    )
    return run(tokens_flat, table_bf, pe_bf)


def kernel(tokens, table):
    tokens_flat = tokens.reshape(-1).astype(jnp.int32)
    table_il = (
        table.reshape(VOCAB, D_MODEL // 32, 2, 16)
        .transpose(0, 1, 3, 2)
        .reshape(-1)
    )
    table_bf = table_il.astype(jnp.bfloat16)
    pe_bf = jnp.asarray(_PE_NP).astype(jnp.bfloat16)
    table_i = lax.bitcast_convert_type(table_bf.reshape(-1, 2), jnp.int32)
    pe_i = lax.bitcast_convert_type(pe_bf.reshape(-1, 2), jnp.int32)
    out = _sc_embed(tokens_flat, table_i, pe_i)
    return out.reshape(BATCH, SEQ, D_MODEL)


# speedup vs baseline: 3.4922x; 1.0608x over previous
import math

import jax
import jax.numpy as jnp
import numpy as np
from jax import lax
from jax.experimental import pallas as pl
from jax.experimental.pallas import tpu as pltpu
from jax.experimental.pallas import tpu_sc as plsc

D_MODEL = 128
MAX_LEN = 1500
VOCAB = 28
BATCH = 256
SEQ = 1024

NC, NS, LANES = 2, 16, 16
NW = NC * NS
TOK_PER_W = BATCH * SEQ // NW
QUARTERS = 4
Q = SEQ // QUARTERS
QD = Q * D_MODEL
SEQ_PER_W = TOK_PER_W // SEQ


def _make_pe_np(max_len, d_model):
    position = np.arange(0, max_len, dtype=np.float32)[:, None]
    div_term = np.exp(
        np.arange(0, d_model, 2).astype(np.float32) * -(math.log(10000.0) / d_model)
    )
    pe = np.zeros((max_len, d_model), dtype=np.float32)
    pe[:, 0::2] = np.sin(position * div_term)
    pe[:, 1::2] = np.cos(position * div_term)
    return pe


def _interleave32_np(x2d):
    n = x2d.shape[0]
    g = x2d.reshape(n, D_MODEL // 32, 2, 16)
    m = np.empty((n, D_MODEL // 32, 16, 2), dtype=x2d.dtype)
    m[..., 0] = g[:, :, 0, :]
    m[..., 1] = g[:, :, 1, :]
    return m.reshape(n * D_MODEL)


_PE_NP = _interleave32_np(_make_pe_np(MAX_LEN, D_MODEL)[:SEQ])


def _sc_embed(tokens_flat, table_bf, pe_bf):
    mesh = plsc.VectorSubcoreMesh(
        core_axis_name="c", subcore_axis_name="s", num_cores=NC, num_subcores=NS
    )

    def body(tok_hbm, table_hbm, pe_hbm, out_hbm,
             table_v, pe0, pe1, idx_v, rows0, rows1,
             sem0, sem1, psem0, psem1):
        wid = lax.axis_index("s") * NC + lax.axis_index("c")
        base = wid * TOK_PER_W
        pes = (pe0, pe1)
        psems = (psem0, psem1)
        pltpu.async_copy(pe_hbm.at[pl.ds(0, QD // 2)], pe0, psem0)
        pltpu.sync_copy(table_hbm, table_v)
        pltpu.sync_copy(tok_hbm.at[pl.ds(base, TOK_PER_W)], idx_v)
        rows = (rows0, rows1)
        sems = (sem0, sem1)

        def compute_chunk(loc, rows_b, pe_v):
            @plsc.parallel_loop(0, Q // LANES, unroll=1)
            def _rb_body(rb):
                tokv = idx_v[pl.ds(loc + rb * LANES, LANES)] * (D_MODEL // 2)
                gbase = rb * (LANES * D_MODEL)
                nj = D_MODEL // 32
                for lane in range(LANES):
                    tbase = tokv[lane]
                    rbase = gbase + lane * D_MODEL
                    pbase = (gbase // 2) + lane * (D_MODEL // 2)
                    tvi = [table_v[pl.ds(tbase + j * LANES, LANES)] for j in range(nj)]
                    pvi = [pe_v[pl.ds(pbase + j * LANES, LANES)] for j in range(nj)]
                    sums = [
                        plsc.bitcast(tvi[j], jnp.bfloat16)
                        + plsc.bitcast(pvi[j], jnp.bfloat16)
                        for j in range(nj)
                    ]
                    for j in range(nj):
                        s0, s1 = plsc.unpack(
                            sums[j],
                            format=plsc.PackFormat.INTERLEAVED,
                            preferred_element_type=jnp.float32,
                        )
                        off = rbase + j * 32
                        rows_b[pl.ds(off, LANES)] = s0
                        rows_b[pl.ds(off + LANES, LANES)] = s1

        for q in range(QUARTERS):
            pe_v = pes[q % 2]
            pltpu.make_async_copy(
                pe_hbm.at[pl.ds(0, QD // 2)], pe_v, psems[q % 2]
            ).wait()
            if q + 1 < QUARTERS:
                pltpu.async_copy(
                    pe_hbm.at[pl.ds((q + 1) * (QD // 2), QD // 2)],
                    pes[(q + 1) % 2],
                    psems[(q + 1) % 2],
                )

            def s2_body(s2, _, q=q, pe_v=pe_v):
                for b in range(2):
                    s = s2 * 2 + b
                    g = base + s * SEQ + q * Q

                    if q == 0:
                        @pl.when(s2 > 0)
                        def _wait(b=b):
                            pltpu.make_async_copy(
                                rows[b], out_hbm.at[pl.ds(0, QD)], sems[b]
                            ).wait()
                    else:
                        pltpu.make_async_copy(
                            rows[b], out_hbm.at[pl.ds(0, QD)], sems[b]
                        ).wait()

                    compute_chunk(s * SEQ + q * Q, rows[b], pe_v)
                    pltpu.async_copy(
                        rows[b], out_hbm.at[pl.ds(g * D_MODEL, QD)], sems[b]
                    )
                return 0

            lax.fori_loop(0, SEQ_PER_W // 2, s2_body, 0)
        for b in range(2):
            pltpu.make_async_copy(
                rows[b], out_hbm.at[pl.ds(0, QD)], sems[b]
            ).wait()

    run = pl.kernel(
        body,
        out_type=jax.ShapeDtypeStruct((BATCH * SEQ * D_MODEL,), jnp.float32),
        mesh=mesh,
        compiler_params=pltpu.CompilerParams(needs_layout_passes=False),
        scratch_types=[
            pltpu.VMEM((VOCAB * D_MODEL // 2,), jnp.int32),
            pltpu.VMEM((QD // 2,), jnp.int32),
            pltpu.VMEM((QD // 2,), jnp.int32),
            pltpu.VMEM((TOK_PER_W,), jnp.int32),
            pltpu.VMEM((QD,), jnp.float32),
            pltpu.VMEM((QD,), jnp.float32),
            pltpu.SemaphoreType.DMA,
            pltpu.SemaphoreType.DMA,
            pltpu.SemaphoreType.DMA,
            pltpu.SemaphoreType.DMA,
        ],
    )
    return run(tokens_flat, table_bf, pe_bf)


def kernel(tokens, table):
    tokens_flat = tokens.reshape(-1).astype(jnp.int32)
    table_il = (
        table.reshape(VOCAB, D_MODEL // 32, 2, 16)
        .transpose(0, 1, 3, 2)
        .reshape(-1)
    )
    table_bf = table_il.astype(jnp.bfloat16)
    pe_bf = jnp.asarray(_PE_NP).astype(jnp.bfloat16)
    table_i = lax.bitcast_convert_type(table_bf.reshape(-1, 2), jnp.int32)
    pe_i = lax.bitcast_convert_type(pe_bf.reshape(-1, 2), jnp.int32)
    out = _sc_embed(tokens_flat, table_i, pe_i)
    return out.reshape(BATCH, SEQ, D_MODEL)
